# Initial kernel scaffold; baseline (speedup 1.0000x reference)
import jax
import jax.numpy as jnp
from jax.experimental import pallas as pl


def kernel(x_l, x_c, C_init_W, C_init_b, W_ih_C, W_hh_C, b_ih_C, b_hh_C, W_ih_L, W_hh_L, b_ih_L, b_hh_L, L_vote_W, L_vote_b, clause_idx, lit_idx, x_l_batch, num_iters):
    raise NotImplementedError("write your pallas kernel here")


# speedup vs baseline: 1.0130x; 1.0130x over previous
import functools

import jax
import jax.numpy as jnp
from jax import lax
from jax.experimental import pallas as pl
from jax.experimental.pallas import tpu as pltpu

D = 128
N_PROB = 80
LITS_PER = 500
HALF = 250
HALF_PAD = 256
NL2 = N_PROB * HALF_PAD * 2
HALF_ROWS = N_PROB * HALF_PAD
N_CLAUSES = 10000
NC_PAD = 10240


def _clause_lstm_body(p0, p1, h, c, wih, whh, b, h_out, c_out):
    gates = (
        jnp.dot(p0[...] + p1[...], wih[...], preferred_element_type=jnp.float32)
        + jnp.dot(h[...], whh[...], preferred_element_type=jnp.float32)
        + b[...]
    )
    i = jax.nn.sigmoid(gates[:, :D])
    f = jax.nn.sigmoid(gates[:, D:2 * D])
    g = jnp.tanh(gates[:, 2 * D:3 * D])
    o = jax.nn.sigmoid(gates[:, 3 * D:])
    c_new = f * c[...] + i * g
    h_out[...] = o * jnp.tanh(c_new)
    c_out[...] = c_new


def _clause_lstm(p0, p1, h, c, wih_t, whh_t, b):
    blk = 1000
    grid = N_CLAUSES // blk
    row = pl.BlockSpec((blk, D), lambda j: (j, 0))
    full = pl.BlockSpec((D, 4 * D), lambda j: (0, 0))
    bias = pl.BlockSpec((1, 4 * D), lambda j: (0, 0))
    return pl.pallas_call(
        _clause_lstm_body,
        grid=(grid,),
        in_specs=[row, row, row, row, full, full, bias],
        out_specs=[row, row],
        out_shape=[
            jax.ShapeDtypeStruct((N_CLAUSES, D), jnp.float32),
            jax.ShapeDtypeStruct((N_CLAUSES, D), jnp.float32),
        ],
    )(p0, p1, h, c, wih_t, whh_t, b)


def _lit_lstm_body(msg, hf, h, c, wm, wf, whh, b, h_out, c_out):
    gates = (
        jnp.dot(msg[...], wm[...], preferred_element_type=jnp.float32)
        + jnp.dot(hf[...], wf[...], preferred_element_type=jnp.float32)
        + jnp.dot(h[...], whh[...], preferred_element_type=jnp.float32)
        + b[...]
    )
    i = jax.nn.sigmoid(gates[:, :D])
    f = jax.nn.sigmoid(gates[:, D:2 * D])
    g = jnp.tanh(gates[:, 2 * D:3 * D])
    o = jax.nn.sigmoid(gates[:, 3 * D:])
    c_new = f * c[...] + i * g
    h_out[...] = o * jnp.tanh(c_new)
    c_out[...] = c_new


def _lit_lstm(msg, h, c, wm_t, wf_t, whh_t, b):
    blk = 512
    grid = NL2 // blk
    row = pl.BlockSpec((blk, D), lambda j: (j, 0))
    flip = pl.BlockSpec((blk, D), lambda j: ((j + grid // 2) % grid, 0))
    full = pl.BlockSpec((D, 4 * D), lambda j: (0, 0))
    bias = pl.BlockSpec((1, 4 * D), lambda j: (0, 0))
    return pl.pallas_call(
        _lit_lstm_body,
        grid=(grid,),
        in_specs=[row, flip, row, row, full, full, full, bias],
        out_specs=[row, row],
        out_shape=[
            jax.ShapeDtypeStruct((NL2, D), jnp.float32),
            jax.ShapeDtypeStruct((NL2, D), jnp.float32),
        ],
    )(msg, h, h, c, wm_t, wf_t, whh_t, b)


_VOTE_PB = 8
_VOTE_ROWS = _VOTE_PB * HALF_PAD


def _vote_body(xa, xb, w, out):
    wv = w[0, :]
    mask = lax.broadcasted_iota(jnp.int32, (_VOTE_PB, HALF_PAD, D), 1) < HALF
    sa = jnp.sum(jnp.where(mask, (xa[...] * wv[None, :]).reshape(
        _VOTE_PB, HALF_PAD, D), 0.0), axis=(1, 2))
    sb = jnp.sum(jnp.where(mask, (xb[...] * wv[None, :]).reshape(
        _VOTE_PB, HALF_PAD, D), 0.0), axis=(1, 2))
    out[...] = jnp.broadcast_to(
        ((sa + sb) * (1.0 / LITS_PER))[:, None], (_VOTE_PB, D))


def _vote(x, w):
    nblk = N_PROB // _VOTE_PB
    blka = pl.BlockSpec((_VOTE_ROWS, D), lambda j: (j, 0))
    blkb = pl.BlockSpec((_VOTE_ROWS, D), lambda j: (nblk + j, 0))
    wspec = pl.BlockSpec((1, D), lambda j: (0, 0))
    return pl.pallas_call(
        _vote_body,
        grid=(nblk,),
        in_specs=[blka, blkb, wspec],
        out_specs=pl.BlockSpec((_VOTE_PB, D), lambda j: (j, 0)),
        out_shape=jax.ShapeDtypeStruct((N_PROB, D), jnp.float32),
    )(x, x, w)


def _msg_to_clauses(x_l_state, lit2, cidx):
    g = jnp.take(x_l_state, lit2, axis=0)
    s = jax.ops.segment_sum(g, cidx, num_segments=N_CLAUSES)
    return s, jnp.zeros_like(s)


def _msg_to_lits(x_c_state, lit2, cidx):
    g = jnp.take(x_c_state, cidx, axis=0)
    return jax.ops.segment_sum(g, lit2, num_segments=NL2)


def kernel(x_l, x_c, C_init_W, C_init_b, W_ih_C, W_hh_C, b_ih_C, b_hh_C,
           W_ih_L, W_hh_L, b_ih_L, b_hh_L, L_vote_W, L_vote_b,
           clause_idx, lit_idx, x_l_batch, num_iters):
    xl4 = x_l.reshape(N_PROB, 2, HALF, D)
    xl4 = jnp.pad(xl4, ((0, 0), (0, 0), (0, HALF_PAD - HALF), (0, 0)))
    x_l2 = jnp.transpose(xl4, (1, 0, 2, 3)).reshape(NL2, D)

    lit = lit_idx.astype(jnp.int32)
    p = lit // LITS_PER
    w = lit % LITS_PER
    lit2 = jnp.where(w < HALF, p * HALF_PAD + w,
                     HALF_ROWS + p * HALF_PAD + (w - HALF))
    cidx = clause_idx.astype(jnp.int32)

    wih_c_t = W_ih_C.T
    whh_c_t = W_hh_C.T
    b_c = (b_ih_C + b_hh_C)[None, :]
    wm_t = W_ih_L[:, :D].T
    wf_t = W_ih_L[:, D:].T
    whh_l_t = W_hh_L.T
    b_l = (b_ih_L + b_hh_L)[None, :]

    c0 = C_init_W[:, 0] + C_init_b
    x_c_state = jnp.tile(c0[None, :], (N_CLAUSES, 1))
    x_c_h = jnp.zeros((N_CLAUSES, D), jnp.float32)
    x_l_state = x_l2
    x_l_h = jnp.zeros((NL2, D), jnp.float32)

    def body(t, carry):
        x_l_state, x_l_h, x_c_state, x_c_h = carry
        p0, p1 = _msg_to_clauses(x_l_state, lit2, cidx)
        x_c_state, x_c_h = _clause_lstm(p0, p1, x_c_state, x_c_h,
                                        wih_c_t, whh_c_t, b_c)
        msg_l = _msg_to_lits(x_c_state, lit2, cidx)
        x_l_state, x_l_h = _lit_lstm(msg_l, x_l_state, x_l_h,
                                     wm_t, wf_t, whh_l_t, b_l)
        return (x_l_state, x_l_h, x_c_state, x_c_h)

    x_l_state, x_l_h, x_c_state, x_c_h = lax.fori_loop(
        0, num_iters, body, (x_l_state, x_l_h, x_c_state, x_c_h))

    logits = _vote(x_l_state, L_vote_W)
    return logits[:, :1] + L_vote_b[None, :]


# speedup vs baseline: 2.9736x; 2.9354x over previous
import functools

import jax
import jax.numpy as jnp
from jax import lax
from jax.experimental import pallas as pl
from jax.experimental.pallas import tpu as pltpu
from jax.experimental.pallas import tpu_sc as plsc

D = 128
N_PROB = 80
LITS_PER = 500
HALF = 250
HALF_PAD = 256
NL2 = N_PROB * HALF_PAD * 2
HALF_ROWS = N_PROB * HALF_PAD
N_CLAUSES = 10000
NC_PAD = 10240


def _clause_lstm_body(p0, p1, h, c, wih, whh, b, h_out, c_out):
    gates = (
        jnp.dot(p0[0] + p1[0], wih[...], preferred_element_type=jnp.float32)
        + jnp.dot(h[...], whh[...], preferred_element_type=jnp.float32)
        + b[...]
    )
    i = jax.nn.sigmoid(gates[:, :D])
    f = jax.nn.sigmoid(gates[:, D:2 * D])
    g = jnp.tanh(gates[:, 2 * D:3 * D])
    o = jax.nn.sigmoid(gates[:, 3 * D:])
    c_new = f * c[...] + i * g
    h_out[...] = o * jnp.tanh(c_new)
    c_out[...] = c_new


def _clause_lstm(p, h, c, wih_t, whh_t, b):
    blk = 1000
    grid = N_CLAUSES // blk
    row = pl.BlockSpec((blk, D), lambda j: (j, 0))
    p0 = pl.BlockSpec((1, blk, D), lambda j: (0, j, 0))
    p1 = pl.BlockSpec((1, blk, D), lambda j: (1, j, 0))
    full = pl.BlockSpec((D, 4 * D), lambda j: (0, 0))
    bias = pl.BlockSpec((1, 4 * D), lambda j: (0, 0))
    return pl.pallas_call(
        _clause_lstm_body,
        grid=(grid,),
        in_specs=[p0, p1, row, row, full, full, bias],
        out_specs=[row, row],
        out_shape=[
            jax.ShapeDtypeStruct((N_CLAUSES, D), jnp.float32),
            jax.ShapeDtypeStruct((N_CLAUSES, D), jnp.float32),
        ],
    )(p, p, h, c, wih_t, whh_t, b)


def _lit_lstm_body(msg, hf, h, c, wm, wf, whh, b, h_out, c_out):
    gates = (
        jnp.dot(msg[...], wm[...], preferred_element_type=jnp.float32)
        + jnp.dot(hf[...], wf[...], preferred_element_type=jnp.float32)
        + jnp.dot(h[...], whh[...], preferred_element_type=jnp.float32)
        + b[...]
    )
    i = jax.nn.sigmoid(gates[:, :D])
    f = jax.nn.sigmoid(gates[:, D:2 * D])
    g = jnp.tanh(gates[:, 2 * D:3 * D])
    o = jax.nn.sigmoid(gates[:, 3 * D:])
    c_new = f * c[...] + i * g
    h_out[...] = o * jnp.tanh(c_new)
    c_out[...] = c_new


def _lit_lstm(msg, h, c, wm_t, wf_t, whh_t, b):
    blk = 512
    grid = NL2 // blk
    row = pl.BlockSpec((blk, D), lambda j: (j, 0))
    flip = pl.BlockSpec((blk, D), lambda j: ((j + grid // 2) % grid, 0))
    full = pl.BlockSpec((D, 4 * D), lambda j: (0, 0))
    bias = pl.BlockSpec((1, 4 * D), lambda j: (0, 0))
    return pl.pallas_call(
        _lit_lstm_body,
        grid=(grid,),
        in_specs=[row, flip, row, row, full, full, full, bias],
        out_specs=[row, row],
        out_shape=[
            jax.ShapeDtypeStruct((NL2, D), jnp.float32),
            jax.ShapeDtypeStruct((NL2, D), jnp.float32),
        ],
    )(msg, h, h, c, wm_t, wf_t, whh_t, b)


_VOTE_PB = 8
_VOTE_ROWS = _VOTE_PB * HALF_PAD


def _vote_body(xa, xb, w, out):
    wv = w[0, :]
    mask = lax.broadcasted_iota(jnp.int32, (_VOTE_PB, HALF_PAD, D), 1) < HALF
    sa = jnp.sum(jnp.where(mask, (xa[...] * wv[None, :]).reshape(
        _VOTE_PB, HALF_PAD, D), 0.0), axis=(1, 2))
    sb = jnp.sum(jnp.where(mask, (xb[...] * wv[None, :]).reshape(
        _VOTE_PB, HALF_PAD, D), 0.0), axis=(1, 2))
    out[...] = jnp.broadcast_to(
        ((sa + sb) * (1.0 / LITS_PER))[:, None], (_VOTE_PB, D))


def _vote(x, w):
    nblk = N_PROB // _VOTE_PB
    blka = pl.BlockSpec((_VOTE_ROWS, D), lambda j: (j, 0))
    blkb = pl.BlockSpec((_VOTE_ROWS, D), lambda j: (nblk + j, 0))
    wspec = pl.BlockSpec((1, D), lambda j: (0, 0))
    return pl.pallas_call(
        _vote_body,
        grid=(nblk,),
        in_specs=[blka, blkb, wspec],
        out_specs=pl.BlockSpec((_VOTE_PB, D), lambda j: (j, 0)),
        out_shape=jax.ShapeDtypeStruct((N_PROB, D), jnp.float32),
    )(x, x, w)


E = 320000
NSC = 2
NTL = 16
BLK = 80
EROWS = E // BLK
STRIPE = NC_PAD // NTL
ZROWS = 40

_SC_MESH = plsc.VectorSubcoreMesh(core_axis_name="c", subcore_axis_name="s")


CHUNK = 25


@functools.partial(
    pl.kernel,
    mesh=_SC_MESH,
    out_type=jax.ShapeDtypeStruct((2 * NC_PAD, D), jnp.float32),
    scratch_types=[
        pltpu.VMEM((CHUNK, BLK), jnp.int32),
        pltpu.VMEM((CHUNK, BLK), jnp.int32),
        pltpu.VMEM((BLK, D), jnp.float32),
        pltpu.VMEM_SHARED((NC_PAD, D), jnp.float32),
        pltpu.SemaphoreType.DMA,
    ],
)
def _msg_c_sc(xl, lit4, cid4, zrows, out, idx_l, idx_c, rows, acc, sem):
    c = lax.axis_index("c")
    s = lax.axis_index("s")
    nch = E // (BLK * CHUNK * 2 * NTL)
    wid = c * NTL + s
    pltpu.sync_copy(zrows, acc.at[pl.ds(s * STRIPE, STRIPE)])
    plsc.subcore_barrier()

    def chunk(ch, carry):
        pltpu.sync_copy(lit4.at[wid, ch], idx_l)
        pltpu.sync_copy(cid4.at[wid, ch], idx_c)

        def blk(b, carry2):
            pltpu.async_copy(xl.at[idx_l.at[b]], rows, sem).wait()
            pltpu.sync_copy(rows, acc.at[idx_c.at[b]], add=True)
            return carry2

        return lax.fori_loop(0, CHUNK, blk, carry)

    lax.fori_loop(0, nch, chunk, 0)
    plsc.subcore_barrier()
    pltpu.sync_copy(acc.at[pl.ds(s * STRIPE, STRIPE)],
                    out.at[pl.ds(c * NC_PAD + s * STRIPE, STRIPE)])


@functools.partial(
    pl.kernel,
    mesh=_SC_MESH,
    out_type=jax.ShapeDtypeStruct((NL2, D), jnp.float32),
    scratch_types=[
        pltpu.VMEM((CHUNK, BLK), jnp.int32),
        pltpu.VMEM((CHUNK, BLK), jnp.int32),
        pltpu.VMEM((CHUNK, BLK), jnp.int32),
        pltpu.VMEM((BLK, D), jnp.float32),
        pltpu.VMEM_SHARED((NC_PAD, D), jnp.float32),
        pltpu.SemaphoreType.DMA,
    ],
)
def _msg_l_sc(xc, lit4, cid4, zrows, out, idx_l, idx_c, idx_s, rows, acc, sem):
    c = lax.axis_index("c")
    s = lax.axis_index("s")
    nch = EROWS // (NTL * CHUNK)
    dump = lax.iota(jnp.int32, 16) * HALF_PAD + HALF

    for rr in range(2):
        base = (c * 2 + rr) * NC_PAD
        pltpu.sync_copy(zrows, acc.at[pl.ds(s * STRIPE, STRIPE)])
        plsc.subcore_barrier()

        def chunk(ch, carry):
            pltpu.sync_copy(lit4.at[s, ch], idx_l)
            pltpu.sync_copy(cid4.at[s, ch], idx_c)

            def tr(i, carry2):
                row = i // (BLK // 16)
                lane = (i % (BLK // 16)) * 16
                v = idx_l[row, pl.ds(lane, 16)]
                local = v - base
                ok = (local >= 0) & (local < NC_PAD)
                idx_s[row, pl.ds(lane, 16)] = jnp.where(ok, local, dump)
                return carry2

            lax.fori_loop(0, CHUNK * (BLK // 16), tr, 0)

            def blk(b, carry2):
                pltpu.async_copy(xc.at[idx_c.at[b]], rows, sem).wait()
                pltpu.sync_copy(rows, acc.at[idx_s.at[b]], add=True)
                return carry2

            return lax.fori_loop(0, CHUNK, blk, carry)

        lax.fori_loop(0, nch, chunk, 0)
        plsc.subcore_barrier()
        pltpu.sync_copy(acc.at[pl.ds(s * STRIPE, STRIPE)],
                        out.at[pl.ds(base + s * STRIPE, STRIPE)])
        plsc.subcore_barrier()


def _msg_to_clauses(x_l_state, lit4c, cid4c, zrows):
    p = _msg_c_sc(x_l_state, lit4c, cid4c, zrows)
    return p.reshape(2, NC_PAD, D)


def _msg_to_lits(x_c_state, lit4l, cid4l, zrows):
    return _msg_l_sc(x_c_state, lit4l, cid4l, zrows)


def kernel(x_l, x_c, C_init_W, C_init_b, W_ih_C, W_hh_C, b_ih_C, b_hh_C,
           W_ih_L, W_hh_L, b_ih_L, b_hh_L, L_vote_W, L_vote_b,
           clause_idx, lit_idx, x_l_batch, num_iters):
    xl4 = x_l.reshape(N_PROB, 2, HALF, D)
    xl4 = jnp.pad(xl4, ((0, 0), (0, 0), (0, HALF_PAD - HALF), (0, 0)))
    x_l2 = jnp.transpose(xl4, (1, 0, 2, 3)).reshape(NL2, D)

    lit = lit_idx.astype(jnp.int32)
    p = lit // LITS_PER
    w = lit % LITS_PER
    lit2 = jnp.where(w < HALF, p * HALF_PAD + w,
                     HALF_ROWS + p * HALF_PAD + (w - HALF))
    cidx = clause_idx.astype(jnp.int32)
    lit4c = lit2.reshape(2 * NTL, -1, CHUNK, BLK)
    cid4c = cidx.reshape(2 * NTL, -1, CHUNK, BLK)
    lit4l = lit2.reshape(NTL, -1, CHUNK, BLK)
    cid4l = cidx.reshape(NTL, -1, CHUNK, BLK)
    zrows = jnp.zeros((STRIPE, D), jnp.float32)

    wih_c_t = W_ih_C.T
    whh_c_t = W_hh_C.T
    b_c = (b_ih_C + b_hh_C)[None, :]
    wm_t = W_ih_L[:, :D].T
    wf_t = W_ih_L[:, D:].T
    whh_l_t = W_hh_L.T
    b_l = (b_ih_L + b_hh_L)[None, :]

    c0 = C_init_W[:, 0] + C_init_b
    x_c_state = jnp.tile(c0[None, :], (N_CLAUSES, 1))
    x_c_h = jnp.zeros((N_CLAUSES, D), jnp.float32)
    x_l_state = x_l2
    x_l_h = jnp.zeros((NL2, D), jnp.float32)

    def body(t, carry):
        x_l_state, x_l_h, x_c_state, x_c_h = carry
        pc = _msg_to_clauses(x_l_state, lit4c, cid4c, zrows)
        x_c_state, x_c_h = _clause_lstm(pc, x_c_state, x_c_h,
                                        wih_c_t, whh_c_t, b_c)
        msg_l = _msg_to_lits(x_c_state, lit4l, cid4l, zrows)
        x_l_state, x_l_h = _lit_lstm(msg_l, x_l_state, x_l_h,
                                     wm_t, wf_t, whh_l_t, b_l)
        return (x_l_state, x_l_h, x_c_state, x_c_h)

    x_l_state, x_l_h, x_c_state, x_c_h = lax.fori_loop(
        0, num_iters, body, (x_l_state, x_l_h, x_c_state, x_c_h))

    logits = _vote(x_l_state, L_vote_W)
    return logits[:, :1] + L_vote_b[None, :]


# speedup vs baseline: 4.0041x; 1.3466x over previous
import functools

import jax
import jax.numpy as jnp
from jax import lax
from jax.experimental import pallas as pl
from jax.experimental.pallas import tpu as pltpu
from jax.experimental.pallas import tpu_sc as plsc

D = 128
N_PROB = 80
LITS_PER = 500
HALF = 250
HALF_PAD = 256
NL2 = N_PROB * HALF_PAD * 2
HALF_ROWS = N_PROB * HALF_PAD
N_CLAUSES = 10000
NC_PAD = 10240


def _clause_lstm_body(p0, p1, h, c, wih, whh, b, h_out, c_out):
    gates = (
        jnp.dot(p0[0] + p1[0], wih[...], preferred_element_type=jnp.float32)
        + jnp.dot(h[...], whh[...], preferred_element_type=jnp.float32)
        + b[...]
    )
    i = jax.nn.sigmoid(gates[:, :D])
    f = jax.nn.sigmoid(gates[:, D:2 * D])
    g = jnp.tanh(gates[:, 2 * D:3 * D])
    o = jax.nn.sigmoid(gates[:, 3 * D:])
    c_new = f * c[...] + i * g
    h_out[...] = o * jnp.tanh(c_new)
    c_out[...] = c_new


def _clause_lstm(p, h, c, wih_t, whh_t, b):
    blk = 1000
    grid = N_CLAUSES // blk
    row = pl.BlockSpec((blk, D), lambda j: (j, 0))
    p0 = pl.BlockSpec((1, blk, D), lambda j: (0, j, 0))
    p1 = pl.BlockSpec((1, blk, D), lambda j: (1, j, 0))
    full = pl.BlockSpec((D, 4 * D), lambda j: (0, 0))
    bias = pl.BlockSpec((1, 4 * D), lambda j: (0, 0))
    return pl.pallas_call(
        _clause_lstm_body,
        grid=(grid,),
        in_specs=[p0, p1, row, row, full, full, bias],
        out_specs=[row, row],
        out_shape=[
            jax.ShapeDtypeStruct((N_CLAUSES, D), jnp.float32),
            jax.ShapeDtypeStruct((N_CLAUSES, D), jnp.float32),
        ],
    )(p, p, h, c, wih_t, whh_t, b)


def _lit_lstm_body(msg, hf, h, c, wm, wf, whh, b, h_out, c_out):
    gates = (
        jnp.dot(msg[...], wm[...], preferred_element_type=jnp.float32)
        + jnp.dot(hf[...], wf[...], preferred_element_type=jnp.float32)
        + jnp.dot(h[...], whh[...], preferred_element_type=jnp.float32)
        + b[...]
    )
    i = jax.nn.sigmoid(gates[:, :D])
    f = jax.nn.sigmoid(gates[:, D:2 * D])
    g = jnp.tanh(gates[:, 2 * D:3 * D])
    o = jax.nn.sigmoid(gates[:, 3 * D:])
    c_new = f * c[...] + i * g
    h_out[...] = o * jnp.tanh(c_new)
    c_out[...] = c_new


def _lit_lstm(msg, h, c, wm_t, wf_t, whh_t, b):
    blk = 512
    grid = NL2 // blk
    row = pl.BlockSpec((blk, D), lambda j: (j, 0))
    flip = pl.BlockSpec((blk, D), lambda j: ((j + grid // 2) % grid, 0))
    full = pl.BlockSpec((D, 4 * D), lambda j: (0, 0))
    bias = pl.BlockSpec((1, 4 * D), lambda j: (0, 0))
    return pl.pallas_call(
        _lit_lstm_body,
        grid=(grid,),
        in_specs=[row, flip, row, row, full, full, full, bias],
        out_specs=[row, row],
        out_shape=[
            jax.ShapeDtypeStruct((NL2, D), jnp.float32),
            jax.ShapeDtypeStruct((NL2, D), jnp.float32),
        ],
    )(msg, h, h, c, wm_t, wf_t, whh_t, b)


_VOTE_PB = 8
_VOTE_ROWS = _VOTE_PB * HALF_PAD


def _vote_body(xa, xb, w, out):
    wv = w[0, :]
    mask = lax.broadcasted_iota(jnp.int32, (_VOTE_PB, HALF_PAD, D), 1) < HALF
    sa = jnp.sum(jnp.where(mask, (xa[...] * wv[None, :]).reshape(
        _VOTE_PB, HALF_PAD, D), 0.0), axis=(1, 2))
    sb = jnp.sum(jnp.where(mask, (xb[...] * wv[None, :]).reshape(
        _VOTE_PB, HALF_PAD, D), 0.0), axis=(1, 2))
    out[...] = jnp.broadcast_to(
        ((sa + sb) * (1.0 / LITS_PER))[:, None], (_VOTE_PB, D))


def _vote(x, w):
    nblk = N_PROB // _VOTE_PB
    blka = pl.BlockSpec((_VOTE_ROWS, D), lambda j: (j, 0))
    blkb = pl.BlockSpec((_VOTE_ROWS, D), lambda j: (nblk + j, 0))
    wspec = pl.BlockSpec((1, D), lambda j: (0, 0))
    return pl.pallas_call(
        _vote_body,
        grid=(nblk,),
        in_specs=[blka, blkb, wspec],
        out_specs=pl.BlockSpec((_VOTE_PB, D), lambda j: (j, 0)),
        out_shape=jax.ShapeDtypeStruct((N_PROB, D), jnp.float32),
    )(x, x, w)


E = 320000
NSC = 2
NTL = 16
BLK = 80
EROWS = E // BLK
STRIPE = NC_PAD // NTL
ZROWS = 40

_SC_MESH = plsc.VectorSubcoreMesh(core_axis_name="c", subcore_axis_name="s")


CHUNK = 25


@functools.partial(
    pl.kernel,
    mesh=_SC_MESH,
    out_type=jax.ShapeDtypeStruct((2 * NC_PAD, D), jnp.float32),
    scratch_types=[
        pltpu.VMEM((CHUNK, BLK), jnp.int32),
        pltpu.VMEM((CHUNK, BLK), jnp.int32),
        pltpu.VMEM((2, BLK, D), jnp.float32),
        pltpu.VMEM_SHARED((NC_PAD, D), jnp.float32),
        pltpu.SemaphoreType.DMA((2,)),
    ],
)
def _msg_c_sc(xl, lit4, cid4, zrows, out, idx_l, idx_c, rows, acc, sem):
    c = lax.axis_index("c")
    s = lax.axis_index("s")
    nch = E // (BLK * CHUNK * 2 * NTL)
    wid = c * NTL + s
    pltpu.sync_copy(zrows, acc.at[pl.ds(s * STRIPE, STRIPE)])
    plsc.subcore_barrier()

    def chunk(ch, carry):
        pltpu.sync_copy(lit4.at[wid, ch], idx_l)
        pltpu.sync_copy(cid4.at[wid, ch], idx_c)
        pltpu.async_copy(xl.at[idx_l.at[0]], rows.at[0], sem.at[0])

        def blk(b, carry2):
            nb = b + 1
            pltpu.async_copy(xl.at[idx_l.at[nb]], rows.at[nb % 2], sem.at[nb % 2])
            pltpu.make_async_copy(xl.at[idx_l.at[b]], rows.at[b % 2],
                                  sem.at[b % 2]).wait()
            pltpu.sync_copy(rows.at[b % 2], acc.at[idx_c.at[b]], add=True)
            return carry2

        lax.fori_loop(0, CHUNK - 1, blk, carry)
        last = CHUNK - 1
        pltpu.make_async_copy(xl.at[idx_l.at[last]], rows.at[last % 2],
                              sem.at[last % 2]).wait()
        pltpu.sync_copy(rows.at[last % 2], acc.at[idx_c.at[last]], add=True)
        return carry

    lax.fori_loop(0, nch, chunk, 0)
    plsc.subcore_barrier()
    pltpu.sync_copy(acc.at[pl.ds(s * STRIPE, STRIPE)],
                    out.at[pl.ds(c * NC_PAD + s * STRIPE, STRIPE)])


@functools.partial(
    pl.kernel,
    mesh=_SC_MESH,
    out_type=jax.ShapeDtypeStruct((NL2, D), jnp.float32),
    scratch_types=[
        pltpu.VMEM((CHUNK, BLK), jnp.int32),
        pltpu.VMEM((CHUNK, BLK), jnp.int32),
        pltpu.VMEM((CHUNK, BLK), jnp.int32),
        pltpu.VMEM((2, BLK, D), jnp.float32),
        pltpu.VMEM_SHARED((NC_PAD, D), jnp.float32),
        pltpu.SemaphoreType.DMA((2,)),
    ],
)
def _msg_l_sc(xc, lit4, cid4, zrows, out, idx_l, idx_c, idx_s, rows, acc, sem):
    c = lax.axis_index("c")
    s = lax.axis_index("s")
    nch = EROWS // (NTL * CHUNK)
    dump = lax.iota(jnp.int32, 16) * HALF_PAD + HALF

    for rr in range(2):
        base = (c * 2 + rr) * NC_PAD
        pltpu.sync_copy(zrows, acc.at[pl.ds(s * STRIPE, STRIPE)])
        plsc.subcore_barrier()

        def chunk(ch, carry):
            pltpu.sync_copy(lit4.at[s, ch], idx_l)
            pltpu.sync_copy(cid4.at[s, ch], idx_c)

            def tr(i, carry2):
                row = i // (BLK // 16)
                lane = (i % (BLK // 16)) * 16
                v = idx_l[row, pl.ds(lane, 16)]
                local = v - base
                ok = (local >= 0) & (local < NC_PAD)
                idx_s[row, pl.ds(lane, 16)] = jnp.where(ok, local, dump)
                return carry2

            lax.fori_loop(0, CHUNK * (BLK // 16), tr, 0)
            pltpu.async_copy(xc.at[idx_c.at[0]], rows.at[0], sem.at[0])

            def blk(b, carry2):
                nb = b + 1
                pltpu.async_copy(xc.at[idx_c.at[nb]], rows.at[nb % 2],
                                 sem.at[nb % 2])
                pltpu.make_async_copy(xc.at[idx_c.at[b]], rows.at[b % 2],
                                      sem.at[b % 2]).wait()
                pltpu.sync_copy(rows.at[b % 2], acc.at[idx_s.at[b]], add=True)
                return carry2

            lax.fori_loop(0, CHUNK - 1, blk, carry)
            last = CHUNK - 1
            pltpu.make_async_copy(xc.at[idx_c.at[last]], rows.at[last % 2],
                                  sem.at[last % 2]).wait()
            pltpu.sync_copy(rows.at[last % 2], acc.at[idx_s.at[last]], add=True)
            return carry

        lax.fori_loop(0, nch, chunk, 0)
        plsc.subcore_barrier()
        pltpu.sync_copy(acc.at[pl.ds(s * STRIPE, STRIPE)],
                        out.at[pl.ds(base + s * STRIPE, STRIPE)])
        plsc.subcore_barrier()


def _msg_to_clauses(x_l_state, lit4c, cid4c, zrows):
    p = _msg_c_sc(x_l_state, lit4c, cid4c, zrows)
    return p.reshape(2, NC_PAD, D)


def _msg_to_lits(x_c_state, lit4l, cid4l, zrows):
    return _msg_l_sc(x_c_state, lit4l, cid4l, zrows)


def kernel(x_l, x_c, C_init_W, C_init_b, W_ih_C, W_hh_C, b_ih_C, b_hh_C,
           W_ih_L, W_hh_L, b_ih_L, b_hh_L, L_vote_W, L_vote_b,
           clause_idx, lit_idx, x_l_batch, num_iters):
    xl4 = x_l.reshape(N_PROB, 2, HALF, D)
    xl4 = jnp.pad(xl4, ((0, 0), (0, 0), (0, HALF_PAD - HALF), (0, 0)))
    x_l2 = jnp.transpose(xl4, (1, 0, 2, 3)).reshape(NL2, D)

    lit = lit_idx.astype(jnp.int32)
    p = lit // LITS_PER
    w = lit % LITS_PER
    lit2 = jnp.where(w < HALF, p * HALF_PAD + w,
                     HALF_ROWS + p * HALF_PAD + (w - HALF))
    cidx = clause_idx.astype(jnp.int32)
    lit4c = lit2.reshape(2 * NTL, -1, CHUNK, BLK)
    cid4c = cidx.reshape(2 * NTL, -1, CHUNK, BLK)
    lit4l = lit2.reshape(NTL, -1, CHUNK, BLK)
    cid4l = cidx.reshape(NTL, -1, CHUNK, BLK)
    zrows = jnp.zeros((STRIPE, D), jnp.float32)

    wih_c_t = W_ih_C.T
    whh_c_t = W_hh_C.T
    b_c = (b_ih_C + b_hh_C)[None, :]
    wm_t = W_ih_L[:, :D].T
    wf_t = W_ih_L[:, D:].T
    whh_l_t = W_hh_L.T
    b_l = (b_ih_L + b_hh_L)[None, :]

    c0 = C_init_W[:, 0] + C_init_b
    x_c_state = jnp.tile(c0[None, :], (N_CLAUSES, 1))
    x_c_h = jnp.zeros((N_CLAUSES, D), jnp.float32)
    x_l_state = x_l2
    x_l_h = jnp.zeros((NL2, D), jnp.float32)

    def body(t, carry):
        x_l_state, x_l_h, x_c_state, x_c_h = carry
        pc = _msg_to_clauses(x_l_state, lit4c, cid4c, zrows)
        x_c_state, x_c_h = _clause_lstm(pc, x_c_state, x_c_h,
                                        wih_c_t, whh_c_t, b_c)
        msg_l = _msg_to_lits(x_c_state, lit4l, cid4l, zrows)
        x_l_state, x_l_h = _lit_lstm(msg_l, x_l_state, x_l_h,
                                     wm_t, wf_t, whh_l_t, b_l)
        return (x_l_state, x_l_h, x_c_state, x_c_h)

    x_l_state, x_l_h, x_c_state, x_c_h = lax.fori_loop(
        0, num_iters, body, (x_l_state, x_l_h, x_c_state, x_c_h))

    logits = _vote(x_l_state, L_vote_W)
    return logits[:, :1] + L_vote_b[None, :]
